# baseline (speedup 1.0000x reference)
import functools

import jax
import jax.numpy as jnp
from jax import lax
from jax.experimental import pallas as pl
from jax.experimental.pallas import tpu as pltpu
from jax.experimental.pallas import tpu_sc as plsc

_info = plsc.get_sparse_core_info()
_NC = _info.num_cores
_NS = _info.num_subcores
_NW = _NC * _NS

_NSLOT = 8
_LP = 56


@functools.partial(jax.jit, static_argnames=("b", "l", "d"))
def _sc_gather(W, idx, *, b, l, d):
    n_per_w = _LP * b // _NW
    rows_per_w = b // _NW
    n_blocks = rows_per_w // (2 * _NSLOT)
    mesh = plsc.VectorSubcoreMesh(core_axis_name="c", subcore_axis_name="s")

    @functools.partial(
        pl.kernel,
        mesh=mesh,
        out_type=jax.ShapeDtypeStruct((b, l, d), jnp.float32),
        scratch_types=[
            pltpu.VMEM((n_per_w,), jnp.int32),
        ]
        + [pltpu.VMEM((l, d), jnp.float32)] * (2 * _NSLOT)
        + [pltpu.SemaphoreType.DMA] * 4,
        compiler_params=pltpu.CompilerParams(use_tc_tiling_on_sc=False),
    )
    def k(table_hbm, idx_hbm, out_hbm, idx_v, *rest):
        bufs = rest[: 2 * _NSLOT]
        gsemA, gsemB, osemA, osemB = rest[2 * _NSLOT :]
        gsems = (gsemA, gsemB)
        osems = (osemA, osemB)
        wid = lax.axis_index("s") * _NC + lax.axis_index("c")
        base = wid * n_per_w
        row0 = wid * rows_per_w
        pltpu.sync_copy(idx_hbm.at[pl.ds(base, n_per_w)], idx_v)

        def gather(j, s, p):
            return pltpu.make_async_copy(
                table_hbm.at[idx_v.at[pl.ds(j * _LP, l)]],
                bufs[p * _NSLOT + s],
                gsems[p],
            )

        def outcp(j, s, p):
            return pltpu.make_async_copy(
                bufs[p * _NSLOT + s],
                out_hbm.at[row0 + j],
                osems[p],
            )

        def fire_gathers(g, p):
            for s in range(_NSLOT):
                gather(g * 2 * _NSLOT + p * _NSLOT + s, s, p).start()

        def drain_block(g, p):
            for s in range(_NSLOT):
                j = g * 2 * _NSLOT + p * _NSLOT + s
                gather(j, s, p).wait()
                outcp(j, s, p).start()

        def drain_outs(g, p):
            for s in range(_NSLOT):
                outcp(g * 2 * _NSLOT + p * _NSLOT + s, s, p).wait()

        fire_gathers(0, 0)
        fire_gathers(0, 1)
        for g in range(n_blocks):
            drain_block(g, 0)
            drain_block(g, 1)
            if g + 1 < n_blocks:
                drain_outs(g, 0)
                fire_gathers(g + 1, 0)
                drain_outs(g, 1)
                fire_gathers(g + 1, 1)
        drain_outs(n_blocks - 1, 0)
        drain_outs(n_blocks - 1, 1)

    return k(W, idx)


def kernel(input_, W):
    b, l = input_.shape
    v, d = W.shape
    idx = jnp.pad(input_, ((0, 0), (0, _LP - l))).reshape(b * _LP)
    return _sc_gather(W, idx, b=b, l=l, d=d)
